# baseline (speedup 1.0000x reference)
import functools

import jax
import jax.numpy as jnp
from jax import lax
from jax.experimental import pallas as pl
from jax.experimental.pallas import tpu as pltpu
from jax.experimental.pallas import tpu_sc as plsc

N_NODES = 10000
D_FEAT = 256
HALF = D_FEAT // 2
N_EDGES = 160000
NC = 2
NS = 16
CHUNK = 128
NB = 2
CPT = 80
ROUNDS = CPT // NB
EPT = CPT * CHUNK
E_PAD = NS * EPT
ACC_ROWS = 10240
ZROWS = 640
OROWS = 640
DUMMY = N_NODES


def _body(x0_hbm, x1_hbm, src_hbm, tgt_hbm, o0_hbm, o1_hbm,
          acc, src_v, rows0, rows1, tgt0, tgt1, gsem0, gsem1, tsem0, tsem1):
    rows = (rows0, rows1)
    tgt = (tgt0, tgt1)
    gsem = (gsem0, gsem1)
    tsem = (tsem0, tsem1)
    c = lax.axis_index("c")
    s = lax.axis_index("s")

    pltpu.sync_copy(src_hbm.at[c, s], src_v)

    zero = jnp.zeros((16,), jnp.float32)

    for xh, oh in ((x0_hbm, o0_hbm), (x1_hbm, o1_hbm)):
        def zrow(i, carry):
            for j in range(HALF // 16):
                rows0[i, pl.ds(j * 16, 16)] = zero
            return carry

        lax.fori_loop(0, CHUNK, zrow, 0)
        base = s * ZROWS
        for k in range(ZROWS // CHUNK):
            pltpu.sync_copy(rows0, acc.at[pl.ds(base + k * CHUNK, CHUNK)])

        for b in range(NB):
            pltpu.async_copy(xh.at[src_v.at[b]], rows[b], gsem[b])
            pltpu.async_copy(tgt_hbm.at[c, s, b], tgt[b], tsem[b])
        plsc.subcore_barrier()

        def rnd(i, carry):
            for b in range(NB):
                k = i * NB + b
                pltpu.make_async_copy(xh.at[src_v.at[k]], rows[b], gsem[b]).wait()
                pltpu.make_async_copy(tgt_hbm.at[c, s, k], tgt[b], tsem[b]).wait()
                pltpu.sync_copy(rows[b], acc.at[tgt[b]], add=True)

                @pl.when(i < ROUNDS - 1)
                def _():
                    pltpu.async_copy(xh.at[src_v.at[k + NB]], rows[b], gsem[b])
                    pltpu.async_copy(tgt_hbm.at[c, s, k + NB], tgt[b], tsem[b])
            return carry

        lax.fori_loop(0, ROUNDS, rnd, 0)
        plsc.subcore_barrier()

        pltpu.sync_copy(acc.at[pl.ds(s * OROWS, OROWS)],
                        oh.at[c, pl.ds(s * OROWS, OROWS)])
        plsc.subcore_barrier()


@jax.jit
def kernel(x, up_index, down_index):
    x = x.astype(jnp.float32)
    x0 = x[:, :HALF]
    x1 = x[:, HALF:]

    pad = E_PAD - N_EDGES
    src = jnp.stack([up_index[0], down_index[0]]).astype(jnp.int32)
    tgt = jnp.stack([up_index[1], down_index[1]]).astype(jnp.int32)
    src = jnp.pad(src, ((0, 0), (0, pad))).reshape(2, NS, CPT, CHUNK)
    tgt = jnp.pad(tgt, ((0, 0), (0, pad)),
                  constant_values=DUMMY).reshape(2, NS, CPT, CHUNK)

    mesh = plsc.VectorSubcoreMesh(core_axis_name="c", subcore_axis_name="s")
    out_t = (jax.ShapeDtypeStruct((2, ACC_ROWS, HALF), jnp.float32),
             jax.ShapeDtypeStruct((2, ACC_ROWS, HALF), jnp.float32))
    kfn = pl.kernel(
        _body,
        out_type=out_t,
        mesh=mesh,
        scratch_types=[
            pltpu.VMEM_SHARED((ACC_ROWS, HALF), jnp.float32),
            pltpu.VMEM((CPT, CHUNK), jnp.int32),
            pltpu.VMEM((CHUNK, HALF), jnp.float32),
            pltpu.VMEM((CHUNK, HALF), jnp.float32),
            pltpu.VMEM((CHUNK,), jnp.int32),
            pltpu.VMEM((CHUNK,), jnp.int32),
            pltpu.SemaphoreType.DMA,
            pltpu.SemaphoreType.DMA,
            pltpu.SemaphoreType.DMA,
            pltpu.SemaphoreType.DMA,
        ],
    )
    o0, o1 = kfn(x0, x1, src, tgt)
    return jnp.concatenate([o0[:, :N_NODES], o1[:, :N_NODES]], axis=-1)


# speedup vs baseline: 1.2968x; 1.2968x over previous
import functools

import jax
import jax.numpy as jnp
from jax import lax
from jax.experimental import pallas as pl
from jax.experimental.pallas import tpu as pltpu
from jax.experimental.pallas import tpu_sc as plsc

N_NODES = 10000
D_FEAT = 256
HALF = D_FEAT // 2
N_EDGES = 160000
NC = 2
NS = 16
CHUNK = 128
CPT = -(-N_EDGES // (NS * CHUNK))
EPT = CPT * CHUNK
E_PAD = NS * EPT
ACC_ROWS = 10240
ZROWS = 640
OROWS = 640
DUMMY = N_NODES
ZB = 64


def _body(x0_hbm, x1_hbm, src_hbm, tgt_hbm, o_hbm,
          acc, zbuf, src_v, tgt_v, rows_v, sem):
    c = lax.axis_index("c")
    s = lax.axis_index("s")

    pltpu.sync_copy(src_hbm.at[c, s], src_v)
    pltpu.sync_copy(tgt_hbm.at[c, s], tgt_v)

    zero = jnp.zeros((16,), jnp.float32)

    def zrow(i, carry):
        for j in range(HALF // 16):
            zbuf[i, pl.ds(j * 16, 16)] = zero
        return carry

    lax.fori_loop(0, ZB, zrow, 0)

    for h, xh in ((0, x0_hbm), (1, x1_hbm)):
        base = s * ZROWS
        for k in range(ZROWS // ZB):
            pltpu.sync_copy(zbuf, acc.at[pl.ds(base + k * ZB, ZB)])
        rem = ZROWS % ZB
        if rem:
            pltpu.sync_copy(zbuf.at[pl.ds(0, rem)],
                            acc.at[pl.ds(base + (ZROWS // ZB) * ZB, rem)])
        plsc.subcore_barrier()

        def chunk(j, carry):
            pltpu.async_copy(xh.at[src_v.at[j]], rows_v, sem).wait()
            pltpu.sync_copy(rows_v, acc.at[tgt_v.at[j]], add=True)
            return carry

        lax.fori_loop(0, CPT, chunk, 0)
        plsc.subcore_barrier()

        pltpu.sync_copy(acc.at[pl.ds(s * OROWS, OROWS)],
                        o_hbm.at[c, pl.ds(s * OROWS, OROWS),
                                 pl.ds(h * HALF, HALF)])
        plsc.subcore_barrier()


@jax.jit
def kernel(x, up_index, down_index):
    x = x.astype(jnp.float32)
    x0 = x[:, :HALF]
    x1 = x[:, HALF:]

    pad = E_PAD - N_EDGES
    src = jnp.stack([up_index[0], down_index[0]]).astype(jnp.int32)
    tgt = jnp.stack([up_index[1], down_index[1]]).astype(jnp.int32)
    src = jnp.pad(src, ((0, 0), (0, pad))).reshape(2, NS, CPT, CHUNK)
    tgt = jnp.pad(tgt, ((0, 0), (0, pad)),
                  constant_values=DUMMY).reshape(2, NS, CPT, CHUNK)

    mesh = plsc.VectorSubcoreMesh(core_axis_name="c", subcore_axis_name="s")
    out_t = jax.ShapeDtypeStruct((2, ACC_ROWS, D_FEAT), jnp.float32)
    kfn = pl.kernel(
        _body,
        out_type=out_t,
        mesh=mesh,
        scratch_types=[
            pltpu.VMEM_SHARED((ACC_ROWS, HALF), jnp.float32),
            pltpu.VMEM((ZB, HALF), jnp.float32),
            pltpu.VMEM((CPT, CHUNK), jnp.int32),
            pltpu.VMEM((CPT, CHUNK), jnp.int32),
            pltpu.VMEM((CHUNK, HALF), jnp.float32),
            pltpu.SemaphoreType.DMA,
        ],
    )
    o = kfn(x0, x1, src, tgt)
    return o[:, :N_NODES]


# speedup vs baseline: 1.3493x; 1.0405x over previous
import functools

import jax
import jax.numpy as jnp
from jax import lax
from jax.experimental import pallas as pl
from jax.experimental.pallas import tpu as pltpu
from jax.experimental.pallas import tpu_sc as plsc

N_NODES = 10000
D_FEAT = 256
HALF = D_FEAT // 2
N_EDGES = 160000
NC = 2
NS = 16
CHUNK = 128
CPT = -(-N_EDGES // (NS * CHUNK))
EPT = CPT * CHUNK
E_PAD = NS * EPT
ACC_ROWS = 10240
ZROWS = 640
OROWS = 640
DUMMY = N_NODES
ZB = 64


def _body(x_hbm, src_hbm, tgt_hbm, o_hbm,
          acc, zbuf, src_v, tgt_v, rows_v, sem):
    c = lax.axis_index("c")
    s = lax.axis_index("s")

    pltpu.sync_copy(src_hbm.at[c, s], src_v)
    pltpu.sync_copy(tgt_hbm.at[c, s], tgt_v)

    zero = jnp.zeros((16,), jnp.float32)

    def zrow(i, carry):
        for j in range(HALF // 16):
            zbuf[i, pl.ds(j * 16, 16)] = zero
        return carry

    lax.fori_loop(0, ZB, zrow, 0)

    for h in (0, 1):
        base = s * ZROWS
        for k in range(ZROWS // ZB):
            pltpu.sync_copy(zbuf, acc.at[pl.ds(base + k * ZB, ZB)])
        rem = ZROWS % ZB
        if rem:
            pltpu.sync_copy(zbuf.at[pl.ds(0, rem)],
                            acc.at[pl.ds(base + (ZROWS // ZB) * ZB, rem)])
        plsc.subcore_barrier()

        def chunk(j, carry):
            pltpu.async_copy(x_hbm.at[src_v.at[j], pl.ds(h * HALF, HALF)],
                             rows_v, sem).wait()
            pltpu.sync_copy(rows_v, acc.at[tgt_v.at[j]], add=True)
            return carry

        lax.fori_loop(0, CPT, chunk, 0)
        plsc.subcore_barrier()

        pltpu.sync_copy(acc.at[pl.ds(s * OROWS, 400)],
                        o_hbm.at[c, pl.ds(s * OROWS, 400),
                                 pl.ds(h * HALF, HALF)])

        @pl.when(s < NS - 1)
        def _():
            pltpu.sync_copy(acc.at[pl.ds(s * OROWS + 400, OROWS - 400)],
                            o_hbm.at[c, pl.ds(s * OROWS + 400, OROWS - 400),
                                     pl.ds(h * HALF, HALF)])
        plsc.subcore_barrier()


@jax.jit
def kernel(x, up_index, down_index):
    x = x.astype(jnp.float32)

    pad = E_PAD - N_EDGES
    src = jnp.stack([up_index[0], down_index[0]]).astype(jnp.int32)
    tgt = jnp.stack([up_index[1], down_index[1]]).astype(jnp.int32)
    src = jnp.pad(src, ((0, 0), (0, pad))).reshape(2, NS, CPT, CHUNK)
    tgt = jnp.pad(tgt, ((0, 0), (0, pad)),
                  constant_values=DUMMY).reshape(2, NS, CPT, CHUNK)

    mesh = plsc.VectorSubcoreMesh(core_axis_name="c", subcore_axis_name="s")
    out_t = jax.ShapeDtypeStruct((2, N_NODES, D_FEAT), jnp.float32)
    kfn = pl.kernel(
        _body,
        out_type=out_t,
        mesh=mesh,
        scratch_types=[
            pltpu.VMEM_SHARED((ACC_ROWS, HALF), jnp.float32),
            pltpu.VMEM((ZB, HALF), jnp.float32),
            pltpu.VMEM((CPT, CHUNK), jnp.int32),
            pltpu.VMEM((CPT, CHUNK), jnp.int32),
            pltpu.VMEM((CHUNK, HALF), jnp.float32),
            pltpu.SemaphoreType.DMA,
        ],
    )
    return kfn(x, src, tgt)
